# Initial kernel scaffold; baseline (speedup 1.0000x reference)
import jax
import jax.numpy as jnp
from jax.experimental import pallas as pl


def kernel(x, edge_index, edge_attr, Wa1, ba1, Wa2, ba2, Wb1, bb1, Wb2, bb2, Wgih, Wghh, bg, Wo1, bo1, Wo2, bo2, Wo3, bo3, Wq1, bq1, Wq2, bq2, Wq3, bq3):
    raise NotImplementedError("write your pallas kernel here")


# speedup vs baseline: 2.8136x; 2.8136x over previous
import functools

import jax
import jax.numpy as jnp
from jax import lax
from jax.experimental import pallas as pl
from jax.experimental.pallas import tpu as pltpu
from jax.experimental.pallas import tpu_sc as plsc

N = 10000
E = 160000
H = 128
CHUNK = 128
NTILES = 32
NCHUNKS = E // CHUNK
CLOOP = (NCHUNKS + NTILES - 1) // NTILES
NPAD = 10240
ROWS_PER_SUB = NPAD // 16
ZROWS = 128

_SC_MESH = plsc.VectorSubcoreMesh(core_axis_name="c", subcore_axis_name="s")


def _sc_msg_body(h_hbm, hb_hbm, src_hbm, dst_hbm, out_hbm,
                 src_v, dst_v, rows_v, hb_v, acc_sh, sem):
    c = lax.axis_index("c")
    s = lax.axis_index("s")
    wid = s * 2 + c

    def _zero(i, carry):
        for j in range(8):
            rows_v[i, pl.ds(j * 16, 16)] = jnp.zeros((16,), jnp.float32)
        return carry
    lax.fori_loop(0, ZROWS, _zero, 0)
    for r in range(ROWS_PER_SUB // ZROWS):
        pltpu.sync_copy(rows_v, acc_sh.at[pl.ds(s * ROWS_PER_SUB + r * ZROWS, ZROWS)])
    plsc.subcore_barrier()

    def _chunk(k, carry):
        cid = wid + k * NTILES

        @pl.when(cid < NCHUNKS)
        def _():
            base = cid * CHUNK
            pltpu.sync_copy(src_hbm.at[pl.ds(base, CHUNK)], src_v)
            pltpu.sync_copy(dst_hbm.at[pl.ds(base, CHUNK)], dst_v)
            pltpu.async_copy(h_hbm.at[src_v], rows_v, sem).wait()
            pltpu.sync_copy(hb_hbm.at[pl.ds(base, CHUNK)], hb_v)

            def _row(r, rc):
                for j in range(8):
                    sl = pl.ds(j * 16, 16)
                    v = rows_v[r, sl] + hb_v[r, sl]
                    rows_v[r, sl] = jnp.maximum(v, 0.0)
                return rc
            lax.fori_loop(0, CHUNK, _row, 0)
            pltpu.sync_copy(rows_v, acc_sh.at[dst_v], add=True)
        return carry

    lax.fori_loop(0, CLOOP, _chunk, 0)
    plsc.subcore_barrier()
    pltpu.sync_copy(acc_sh.at[pl.ds(s * ROWS_PER_SUB, ROWS_PER_SUB)],
                    out_hbm.at[c, pl.ds(s * ROWS_PER_SUB, ROWS_PER_SUB)])


_sc_msg = functools.partial(
    pl.kernel,
    out_type=jax.ShapeDtypeStruct((2, NPAD, H), jnp.float32),
    mesh=_SC_MESH,
    scratch_types=[
        pltpu.VMEM((CHUNK,), jnp.int32),
        pltpu.VMEM((CHUNK,), jnp.int32),
        pltpu.VMEM((CHUNK, H), jnp.float32),
        pltpu.VMEM((CHUNK, H), jnp.float32),
        pltpu.VMEM_SHARED((NPAD, H), jnp.float32),
        pltpu.SemaphoreType.DMA,
    ],
)(_sc_msg_body)


def _sc_pair_body(h_hbm, src_hbm, dst_hbm, hs_hbm, hd_hbm,
                  src_v, dst_v, rows_s, rows_d, sem_s, sem_d):
    c = lax.axis_index("c")
    s = lax.axis_index("s")
    wid = s * 2 + c

    def _chunk(k, carry):
        cid = wid + k * NTILES

        @pl.when(cid < NCHUNKS)
        def _():
            base = cid * CHUNK
            pltpu.sync_copy(src_hbm.at[pl.ds(base, CHUNK)], src_v)
            pltpu.sync_copy(dst_hbm.at[pl.ds(base, CHUNK)], dst_v)
            cp_s = pltpu.async_copy(h_hbm.at[src_v], rows_s, sem_s)
            cp_d = pltpu.async_copy(h_hbm.at[dst_v], rows_d, sem_d)
            cp_s.wait()
            cp_d.wait()
            pltpu.sync_copy(rows_s, hs_hbm.at[pl.ds(base, CHUNK)])
            pltpu.sync_copy(rows_d, hd_hbm.at[pl.ds(base, CHUNK)])
        return carry

    lax.fori_loop(0, CLOOP, _chunk, 0)


_sc_pair = functools.partial(
    pl.kernel,
    out_type=(jax.ShapeDtypeStruct((E, H), jnp.float32),
              jax.ShapeDtypeStruct((E, H), jnp.float32)),
    mesh=_SC_MESH,
    scratch_types=[
        pltpu.VMEM((CHUNK,), jnp.int32),
        pltpu.VMEM((CHUNK,), jnp.int32),
        pltpu.VMEM((CHUNK, H), jnp.float32),
        pltpu.VMEM((CHUNK, H), jnp.float32),
        pltpu.SemaphoreType.DMA,
        pltpu.SemaphoreType.DMA,
    ],
)(_sc_pair_body)


def _mlp2_kern(x_ref, w1_ref, b1_ref, w2_ref, b2_ref, o_ref):
    t = jnp.maximum(
        jnp.dot(x_ref[...], w1_ref[...], preferred_element_type=jnp.float32)
        + b1_ref[...], 0.0)
    o_ref[...] = (jnp.dot(t, w2_ref[...], preferred_element_type=jnp.float32)
                  + b2_ref[...])


def _mlp2(x, w1, b1, w2, b2, blk):
    m, din = x.shape
    dmid = w1.shape[1]
    dout = w2.shape[1]
    grid = m // blk
    return pl.pallas_call(
        _mlp2_kern,
        grid=(grid,),
        in_specs=[
            pl.BlockSpec((blk, din), lambda i: (i, 0)),
            pl.BlockSpec((din, dmid), lambda i: (0, 0)),
            pl.BlockSpec((1, dmid), lambda i: (0, 0)),
            pl.BlockSpec((dmid, dout), lambda i: (0, 0)),
            pl.BlockSpec((1, dout), lambda i: (0, 0)),
        ],
        out_specs=pl.BlockSpec((blk, dout), lambda i: (i, 0)),
        out_shape=jax.ShapeDtypeStruct((m, dout), jnp.float32),
    )(x, w1, b1.reshape(1, -1), w2, b2.reshape(1, -1))


def _gru_kern(p_ref, h_ref, wih_ref, whh_ref, bg_ref, o_ref):
    msg = p_ref[0] + p_ref[1]
    gi = jnp.dot(msg, wih_ref[...], preferred_element_type=jnp.float32) + bg_ref[...]
    gh = jnp.dot(h_ref[...], whh_ref[...], preferred_element_type=jnp.float32)
    z = jax.nn.sigmoid(gi[:, :H] + gh[:, :H])
    r = jax.nn.sigmoid(gi[:, H:2 * H] + gh[:, H:2 * H])
    n = jnp.tanh(gi[:, 2 * H:] + r * gh[:, 2 * H:])
    o_ref[...] = (1.0 - z) * n + z * h_ref[...]


def _gru(partials, h, wih, whh, bg, blk=2000):
    grid = N // blk
    return pl.pallas_call(
        _gru_kern,
        grid=(grid,),
        in_specs=[
            pl.BlockSpec((2, blk, H), lambda i: (0, i, 0)),
            pl.BlockSpec((blk, H), lambda i: (i, 0)),
            pl.BlockSpec((H, 3 * H), lambda i: (0, 0)),
            pl.BlockSpec((H, 3 * H), lambda i: (0, 0)),
            pl.BlockSpec((1, 3 * H), lambda i: (0, 0)),
        ],
        out_specs=pl.BlockSpec((blk, H), lambda i: (i, 0)),
        out_shape=jax.ShapeDtypeStruct((N, H), jnp.float32),
    )(partials, h, wih, whh, bg.reshape(1, -1))


def _bond_out_kern(hs_ref, hd_ref, wt_ref, wb_ref, b1_ref, w2_ref, b2_ref,
                   w3_ref, b3_ref, o_ref):
    t = jnp.maximum(
        jnp.dot(hs_ref[...], wt_ref[...], preferred_element_type=jnp.float32)
        + jnp.dot(hd_ref[...], wb_ref[...], preferred_element_type=jnp.float32)
        + b1_ref[...], 0.0)
    t = jnp.maximum(
        jnp.dot(t, w2_ref[...], preferred_element_type=jnp.float32)
        + b2_ref[...], 0.0)
    o_ref[...] = (jnp.dot(t, w3_ref[...], preferred_element_type=jnp.float32)
                  + b3_ref[...])


def _bond_out(hs, hd, wo1, bo1, wo2, bo2, wo3, bo3, blk=8000):
    grid = E // blk
    dmid = wo1.shape[1]
    dmid2 = wo2.shape[1]
    k = wo3.shape[1]
    return pl.pallas_call(
        _bond_out_kern,
        grid=(grid,),
        in_specs=[
            pl.BlockSpec((blk, H), lambda i: (i, 0)),
            pl.BlockSpec((blk, H), lambda i: (i, 0)),
            pl.BlockSpec((H, dmid), lambda i: (0, 0)),
            pl.BlockSpec((H, dmid), lambda i: (0, 0)),
            pl.BlockSpec((1, dmid), lambda i: (0, 0)),
            pl.BlockSpec((dmid, dmid2), lambda i: (0, 0)),
            pl.BlockSpec((1, dmid2), lambda i: (0, 0)),
            pl.BlockSpec((dmid2, k), lambda i: (0, 0)),
            pl.BlockSpec((1, k), lambda i: (0, 0)),
        ],
        out_specs=pl.BlockSpec((blk, k), lambda i: (i, 0)),
        out_shape=jax.ShapeDtypeStruct((E, k), jnp.float32),
    )(hs, hd, wo1[:H], wo1[H:], bo1.reshape(1, -1), wo2, bo2.reshape(1, -1),
      wo3, bo3.reshape(1, -1))


def _graph_out_kern(h_ref, w1_ref, b1_ref, w2_ref, b2_ref, w3_ref, b3_ref,
                    o_ref):
    g = jnp.sum(h_ref[...], axis=0, keepdims=True) * (1.0 / N)
    t = jnp.maximum(
        jnp.dot(g, w1_ref[...], preferred_element_type=jnp.float32)
        + b1_ref[...], 0.0)
    t = jnp.maximum(
        jnp.dot(t, w2_ref[...], preferred_element_type=jnp.float32)
        + b2_ref[...], 0.0)
    o_ref[...] = (jnp.dot(t, w3_ref[...], preferred_element_type=jnp.float32)
                  + b3_ref[...])


def _graph_out(h, wq1, bq1, wq2, bq2, wq3, bq3):
    return pl.pallas_call(
        _graph_out_kern,
        out_shape=jax.ShapeDtypeStruct((1, wq3.shape[1]), jnp.float32),
    )(h, wq1, bq1.reshape(1, -1), wq2, bq2.reshape(1, -1), wq3,
      bq3.reshape(1, -1))


def kernel(x, edge_index, edge_attr, Wa1, ba1, Wa2, ba2, Wb1, bb1, Wb2, bb2,
           Wgih, Wghh, bg, Wo1, bo1, Wo2, bo2, Wo3, bo3, Wq1, bq1, Wq2, bq2,
           Wq3, bq3):
    src = edge_index[0]
    dst = edge_index[1]

    wa2p = jnp.pad(Wa2, ((0, 0), (0, 1)))
    ba2p = jnp.pad(ba2, (0, 1))
    h = _mlp2(x, Wa1, ba1, wa2p, ba2p, blk=2000)

    hb = _mlp2(edge_attr, Wb1, bb1, Wb2, bb2, blk=8000)

    for _ in range(3):
        partials = _sc_msg(h, hb, src, dst)
        h = _gru(partials, h, Wgih, Wghh, bg)

    hs, hd = _sc_pair(h, src, dst)
    bond_scores = _bond_out(hs, hd, Wo1, bo1, Wo2, bo2, Wo3, bo3)

    graph_scores = _graph_out(h, Wq1, bq1, Wq2, bq2, Wq3, bq3)
    return bond_scores, graph_scores.reshape(-1)


# speedup vs baseline: 4.3817x; 1.5574x over previous
import functools

import jax
import jax.numpy as jnp
from jax import lax
from jax.experimental import pallas as pl
from jax.experimental.pallas import tpu as pltpu
from jax.experimental.pallas import tpu_sc as plsc

N = 10000
E = 160000
H = 128
CHUNK = 64
NTILES = 32
NCHUNKS = E // CHUNK
CLOOP = 80
NPAD = 10112
ROWS_PER_SUB = NPAD // 16

_SC_MESH = plsc.VectorSubcoreMesh(core_axis_name="c", subcore_axis_name="s")


def _sc_msg_body(h_hbm, hb_hbm, src_hbm, dst_hbm, zero_hbm, out_hbm,
                 s0, s1, s2, s3, d0, d1, d2, d3, r0, r1, r2, r3, b0, b1,
                 acc_sh,
                 si0, si1, si2, si3, sg0, sg1, sh0, sh1, ss0, ss1):
    srcs = [s0, s1, s2, s3]
    dsts = [d0, d1, d2, d3]
    rows = [r0, r1, r2, r3]
    hbb = [b0, b1]
    sem_i = [si0, si1, si2, si3]
    sem_g = [sg0, sg1]
    sem_h = [sh0, sh1]
    sem_s = [ss0, ss1]

    c = lax.axis_index("c")
    s = lax.axis_index("s")
    wid = s * 2 + c

    pltpu.sync_copy(zero_hbm, acc_sh.at[pl.ds(s * ROWS_PER_SUB, ROWS_PER_SUB)])
    plsc.subcore_barrier()

    def _cid(j):
        return wid + j * NTILES

    def _ok(j):
        return jnp.logical_and(j >= 0, _cid(j) < NCHUNKS)

    def issue_idx(j, m):
        @pl.when(_ok(j))
        def _():
            base = _cid(j) * CHUNK
            pltpu.async_copy(src_hbm.at[pl.ds(base, CHUNK)], srcs[m], sem_i[m])
            pltpu.async_copy(dst_hbm.at[pl.ds(base, CHUNK)], dsts[m], sem_i[m])

    def wait_idx(j, m):
        @pl.when(_ok(j))
        def _():
            base = _cid(j) * CHUNK
            pltpu.make_async_copy(src_hbm.at[pl.ds(base, CHUNK)], srcs[m], sem_i[m]).wait()
            pltpu.make_async_copy(dst_hbm.at[pl.ds(base, CHUNK)], dsts[m], sem_i[m]).wait()

    def issue_fetch(j, m, p):
        @pl.when(_ok(j))
        def _():
            base = _cid(j) * CHUNK
            pltpu.async_copy(h_hbm.at[srcs[m]], rows[m], sem_g[p])
            pltpu.async_copy(hb_hbm.at[pl.ds(base, CHUNK)], hbb[p], sem_h[p])

    def wait_fetch(j, m, p):
        @pl.when(_ok(j))
        def _():
            base = _cid(j) * CHUNK
            pltpu.make_async_copy(h_hbm.at[srcs[m]], rows[m], sem_g[p]).wait()
            pltpu.make_async_copy(hb_hbm.at[pl.ds(base, CHUNK)], hbb[p], sem_h[p]).wait()

    def compute(j, m, p):
        @pl.when(_ok(j))
        def _():
            def _row(r, rc):
                for q in range(8):
                    sl = pl.ds(q * 16, 16)
                    v = rows[m][r, sl] + hbb[p][r, sl]
                    rows[m][r, sl] = jnp.maximum(v, 0.0)
                return rc
            lax.fori_loop(0, CHUNK, _row, 0)

    def issue_scatter(j, m, p):
        @pl.when(_ok(j))
        def _():
            pltpu.async_copy(rows[m], acc_sh.at[dsts[m]], sem_s[p], add=True)

    def wait_scatter(j, m, p):
        @pl.when(_ok(j))
        def _():
            pltpu.make_async_copy(rows[m], acc_sh.at[dsts[m]], sem_s[p]).wait()

    issue_idx(0, 0)
    issue_idx(1, 1)
    wait_idx(0, 0)
    issue_fetch(0, 0, 0)

    def _outer(t, carry):
        for r in range(4):
            k = 4 * t + r
            wait_scatter(k - 2, (r + 2) % 4, r % 2)
            wait_idx(k + 1, (r + 1) % 4)
            issue_fetch(k + 1, (r + 1) % 4, (r + 1) % 2)
            issue_idx(k + 2, (r + 2) % 4)
            wait_fetch(k, r % 4, r % 2)
            compute(k, r % 4, r % 2)
            issue_scatter(k, r % 4, r % 2)
        return carry

    lax.fori_loop(0, CLOOP // 4, _outer, 0)
    wait_scatter(CLOOP - 2, (CLOOP - 2) % 4, CLOOP % 2)
    wait_scatter(CLOOP - 1, (CLOOP - 1) % 4, (CLOOP + 1) % 2)

    plsc.subcore_barrier()
    pltpu.sync_copy(acc_sh.at[pl.ds(s * ROWS_PER_SUB, ROWS_PER_SUB)],
                    out_hbm.at[c, pl.ds(s * ROWS_PER_SUB, ROWS_PER_SUB)])


_sc_msg = functools.partial(
    pl.kernel,
    out_type=jax.ShapeDtypeStruct((2, NPAD, H), jnp.float32),
    mesh=_SC_MESH,
    scratch_types=(
        [pltpu.VMEM((CHUNK,), jnp.int32)] * 8
        + [pltpu.VMEM((CHUNK, H), jnp.float32)] * 6
        + [pltpu.VMEM_SHARED((NPAD, H), jnp.float32)]
        + [pltpu.SemaphoreType.DMA] * 10
    ),
)(_sc_msg_body)


def _sc_pair_body(h_hbm, src_hbm, dst_hbm, hs_hbm, hd_hbm,
                  s0, s1, s2, s3, d0, d1, d2, d3,
                  rs0, rs1, rs2, rs3, rd0, rd1, rd2, rd3,
                  si0, si1, si2, si3, sg0, sg1, sw0, sw1):
    srcs = [s0, s1, s2, s3]
    dsts = [d0, d1, d2, d3]
    rows_s = [rs0, rs1, rs2, rs3]
    rows_d = [rd0, rd1, rd2, rd3]
    sem_i = [si0, si1, si2, si3]
    sem_g = [sg0, sg1]
    sem_w = [sw0, sw1]

    c = lax.axis_index("c")
    s = lax.axis_index("s")
    wid = s * 2 + c

    def _cid(j):
        return wid + j * NTILES

    def _ok(j):
        return jnp.logical_and(j >= 0, _cid(j) < NCHUNKS)

    def issue_idx(j, m):
        @pl.when(_ok(j))
        def _():
            base = _cid(j) * CHUNK
            pltpu.async_copy(src_hbm.at[pl.ds(base, CHUNK)], srcs[m], sem_i[m])
            pltpu.async_copy(dst_hbm.at[pl.ds(base, CHUNK)], dsts[m], sem_i[m])

    def wait_idx(j, m):
        @pl.when(_ok(j))
        def _():
            base = _cid(j) * CHUNK
            pltpu.make_async_copy(src_hbm.at[pl.ds(base, CHUNK)], srcs[m], sem_i[m]).wait()
            pltpu.make_async_copy(dst_hbm.at[pl.ds(base, CHUNK)], dsts[m], sem_i[m]).wait()

    def issue_gather(j, m, p):
        @pl.when(_ok(j))
        def _():
            pltpu.async_copy(h_hbm.at[srcs[m]], rows_s[m], sem_g[p])
            pltpu.async_copy(h_hbm.at[dsts[m]], rows_d[m], sem_g[p])

    def wait_gather(j, m, p):
        @pl.when(_ok(j))
        def _():
            pltpu.make_async_copy(h_hbm.at[srcs[m]], rows_s[m], sem_g[p]).wait()
            pltpu.make_async_copy(h_hbm.at[dsts[m]], rows_d[m], sem_g[p]).wait()

    def issue_write(j, m, p):
        @pl.when(_ok(j))
        def _():
            base = _cid(j) * CHUNK
            pltpu.async_copy(rows_s[m], hs_hbm.at[pl.ds(base, CHUNK)], sem_w[p])
            pltpu.async_copy(rows_d[m], hd_hbm.at[pl.ds(base, CHUNK)], sem_w[p])

    def wait_write(j, m, p):
        @pl.when(_ok(j))
        def _():
            base = _cid(j) * CHUNK
            pltpu.make_async_copy(rows_s[m], hs_hbm.at[pl.ds(base, CHUNK)], sem_w[p]).wait()
            pltpu.make_async_copy(rows_d[m], hd_hbm.at[pl.ds(base, CHUNK)], sem_w[p]).wait()

    issue_idx(0, 0)
    issue_idx(1, 1)
    wait_idx(0, 0)
    issue_gather(0, 0, 0)

    def _outer(t, carry):
        for r in range(4):
            k = 4 * t + r
            wait_write(k - 2, (r + 2) % 4, r % 2)
            wait_idx(k + 1, (r + 1) % 4)
            issue_gather(k + 1, (r + 1) % 4, (r + 1) % 2)
            issue_idx(k + 2, (r + 2) % 4)
            wait_gather(k, r % 4, r % 2)
            issue_write(k, r % 4, r % 2)
        return carry

    lax.fori_loop(0, CLOOP // 4, _outer, 0)
    wait_write(CLOOP - 2, (CLOOP - 2) % 4, CLOOP % 2)
    wait_write(CLOOP - 1, (CLOOP - 1) % 4, (CLOOP + 1) % 2)


_sc_pair = functools.partial(
    pl.kernel,
    out_type=(jax.ShapeDtypeStruct((E, H), jnp.float32),
              jax.ShapeDtypeStruct((E, H), jnp.float32)),
    mesh=_SC_MESH,
    scratch_types=(
        [pltpu.VMEM((CHUNK,), jnp.int32)] * 8
        + [pltpu.VMEM((CHUNK, H), jnp.float32)] * 8
        + [pltpu.SemaphoreType.DMA] * 8
    ),
)(_sc_pair_body)


def _mlp2_kern(x_ref, w1_ref, b1_ref, w2_ref, b2_ref, o_ref):
    t = jnp.maximum(
        jnp.dot(x_ref[...], w1_ref[...], preferred_element_type=jnp.float32)
        + b1_ref[...], 0.0)
    o_ref[...] = (jnp.dot(t, w2_ref[...], preferred_element_type=jnp.float32)
                  + b2_ref[...])


def _mlp2(x, w1, b1, w2, b2, blk):
    m, din = x.shape
    dmid = w1.shape[1]
    dout = w2.shape[1]
    grid = m // blk
    return pl.pallas_call(
        _mlp2_kern,
        grid=(grid,),
        in_specs=[
            pl.BlockSpec((blk, din), lambda i: (i, 0)),
            pl.BlockSpec((din, dmid), lambda i: (0, 0)),
            pl.BlockSpec((1, dmid), lambda i: (0, 0)),
            pl.BlockSpec((dmid, dout), lambda i: (0, 0)),
            pl.BlockSpec((1, dout), lambda i: (0, 0)),
        ],
        out_specs=pl.BlockSpec((blk, dout), lambda i: (i, 0)),
        out_shape=jax.ShapeDtypeStruct((m, dout), jnp.float32),
    )(x, w1, b1.reshape(1, -1), w2, b2.reshape(1, -1))


def _gru_kern(p_ref, h_ref, wih_ref, whh_ref, bg_ref, o_ref):
    msg = p_ref[0] + p_ref[1]
    gi = jnp.dot(msg, wih_ref[...], preferred_element_type=jnp.float32) + bg_ref[...]
    gh = jnp.dot(h_ref[...], whh_ref[...], preferred_element_type=jnp.float32)
    z = jax.nn.sigmoid(gi[:, :H] + gh[:, :H])
    r = jax.nn.sigmoid(gi[:, H:2 * H] + gh[:, H:2 * H])
    n = jnp.tanh(gi[:, 2 * H:] + r * gh[:, 2 * H:])
    o_ref[...] = (1.0 - z) * n + z * h_ref[...]


def _gru(partials, h, wih, whh, bg, blk=2000):
    grid = N // blk
    return pl.pallas_call(
        _gru_kern,
        grid=(grid,),
        in_specs=[
            pl.BlockSpec((2, blk, H), lambda i: (0, i, 0)),
            pl.BlockSpec((blk, H), lambda i: (i, 0)),
            pl.BlockSpec((H, 3 * H), lambda i: (0, 0)),
            pl.BlockSpec((H, 3 * H), lambda i: (0, 0)),
            pl.BlockSpec((1, 3 * H), lambda i: (0, 0)),
        ],
        out_specs=pl.BlockSpec((blk, H), lambda i: (i, 0)),
        out_shape=jax.ShapeDtypeStruct((N, H), jnp.float32),
    )(partials, h, wih, whh, bg.reshape(1, -1))


def _bond_out_kern(hs_ref, hd_ref, wt_ref, wb_ref, b1_ref, w2_ref, b2_ref,
                   w3_ref, b3_ref, o_ref):
    t = jnp.maximum(
        jnp.dot(hs_ref[...], wt_ref[...], preferred_element_type=jnp.float32)
        + jnp.dot(hd_ref[...], wb_ref[...], preferred_element_type=jnp.float32)
        + b1_ref[...], 0.0)
    t = jnp.maximum(
        jnp.dot(t, w2_ref[...], preferred_element_type=jnp.float32)
        + b2_ref[...], 0.0)
    o_ref[...] = (jnp.dot(t, w3_ref[...], preferred_element_type=jnp.float32)
                  + b3_ref[...])


def _bond_out(hs, hd, wo1, bo1, wo2, bo2, wo3, bo3, blk=8000):
    grid = E // blk
    dmid = wo1.shape[1]
    dmid2 = wo2.shape[1]
    k = wo3.shape[1]
    return pl.pallas_call(
        _bond_out_kern,
        grid=(grid,),
        in_specs=[
            pl.BlockSpec((blk, H), lambda i: (i, 0)),
            pl.BlockSpec((blk, H), lambda i: (i, 0)),
            pl.BlockSpec((H, dmid), lambda i: (0, 0)),
            pl.BlockSpec((H, dmid), lambda i: (0, 0)),
            pl.BlockSpec((1, dmid), lambda i: (0, 0)),
            pl.BlockSpec((dmid, dmid2), lambda i: (0, 0)),
            pl.BlockSpec((1, dmid2), lambda i: (0, 0)),
            pl.BlockSpec((dmid2, k), lambda i: (0, 0)),
            pl.BlockSpec((1, k), lambda i: (0, 0)),
        ],
        out_specs=pl.BlockSpec((blk, k), lambda i: (i, 0)),
        out_shape=jax.ShapeDtypeStruct((E, k), jnp.float32),
    )(hs, hd, wo1[:H], wo1[H:], bo1.reshape(1, -1), wo2, bo2.reshape(1, -1),
      wo3, bo3.reshape(1, -1))


def _graph_out_kern(h_ref, w1_ref, b1_ref, w2_ref, b2_ref, w3_ref, b3_ref,
                    o_ref):
    g = jnp.sum(h_ref[...], axis=0, keepdims=True) * (1.0 / N)
    t = jnp.maximum(
        jnp.dot(g, w1_ref[...], preferred_element_type=jnp.float32)
        + b1_ref[...], 0.0)
    t = jnp.maximum(
        jnp.dot(t, w2_ref[...], preferred_element_type=jnp.float32)
        + b2_ref[...], 0.0)
    o_ref[...] = (jnp.dot(t, w3_ref[...], preferred_element_type=jnp.float32)
                  + b3_ref[...])


def _graph_out(h, wq1, bq1, wq2, bq2, wq3, bq3):
    return pl.pallas_call(
        _graph_out_kern,
        out_shape=jax.ShapeDtypeStruct((1, wq3.shape[1]), jnp.float32),
    )(h, wq1, bq1.reshape(1, -1), wq2, bq2.reshape(1, -1), wq3,
      bq3.reshape(1, -1))


def kernel(x, edge_index, edge_attr, Wa1, ba1, Wa2, ba2, Wb1, bb1, Wb2, bb2,
           Wgih, Wghh, bg, Wo1, bo1, Wo2, bo2, Wo3, bo3, Wq1, bq1, Wq2, bq2,
           Wq3, bq3):
    src = edge_index[0]
    dst = edge_index[1]

    wa2p = jnp.pad(Wa2, ((0, 0), (0, 1)))
    ba2p = jnp.pad(ba2, (0, 1))
    h = _mlp2(x, Wa1, ba1, wa2p, ba2p, blk=2000)

    hb = _mlp2(edge_attr, Wb1, bb1, Wb2, bb2, blk=8000)

    zrows = jnp.zeros((ROWS_PER_SUB, H), jnp.float32)
    for _ in range(3):
        partials = _sc_msg(h, hb, src, dst, zrows)
        h = _gru(partials, h, Wgih, Wghh, bg)

    hs, hd = _sc_pair(h, src, dst)
    bond_scores = _bond_out(hs, hd, Wo1, bo1, Wo2, bo2, Wo3, bo3)

    graph_scores = _graph_out(h, Wq1, bq1, Wq2, bq2, Wq3, bq3)
    return bond_scores, graph_scores.reshape(-1)
